# Initial kernel scaffold; baseline (speedup 1.0000x reference)
import jax
import jax.numpy as jnp
from jax.experimental import pallas as pl


def kernel(z, edge_index, pos_edge_index):
    raise NotImplementedError("write your pallas kernel here")


# speedup vs baseline: 44.3704x; 44.3704x over previous
import jax
import jax.numpy as jnp
from jax import lax
from jax.experimental import pallas as pl
from jax.experimental.pallas import tpu as pltpu
from jax.experimental.pallas import tpu_sc as plsc

N_NODES = 10000
N_EDGES = 320000
L = 16
NC = 2
NS = 16
NPAD = 10240
SLICE = NPAD // NS
E_HIST = N_EDGES // NS
E_OUT = N_EDGES // (NC * NS)


def _sc_body(pos_hbm, tail_hbm, out_hbm, hist_v, idx_v, staged_v, red_v,
             outbuf_v, rows_sh, deg_sh):
    cid = lax.axis_index("c")
    sid = lax.axis_index("s")
    wid = sid * NC + cid

    ones = jnp.ones((L,), jnp.float32)
    zeros = jnp.zeros((L,), jnp.float32)

    def _zero(i, c):
        hist_v[pl.ds(i * L, L)] = zeros
        return c
    lax.fori_loop(0, NPAD // L, _zero, 0)

    pltpu.sync_copy(pos_hbm.at[pl.ds(sid * E_HIST, E_HIST)], idx_v)

    def _hist(e, c):
        plsc.addupdate_scatter(hist_v, [idx_v[pl.ds(e * L, L)]], ones)
        return c
    lax.fori_loop(0, E_HIST // L, _hist, 0)

    pltpu.sync_copy(hist_v, rows_sh.at[sid])
    plsc.subcore_barrier()

    col = sid * SLICE
    for r in range(NS):
        pltpu.sync_copy(rows_sh.at[r, pl.ds(col, SLICE)], staged_v.at[r])

    def _reduce(j, c):
        acc = staged_v[0, pl.ds(j * L, L)]
        for r in range(1, NS):
            acc = acc + staged_v[r, pl.ds(j * L, L)]
        red_v[pl.ds(j * L, L)] = acc
        return c
    lax.fori_loop(0, SLICE // L, _reduce, 0)
    pltpu.sync_copy(red_v, deg_sh.at[pl.ds(col, SLICE)])
    plsc.subcore_barrier()

    pltpu.sync_copy(deg_sh, hist_v)
    gbase = wid * E_OUT
    pltpu.sync_copy(tail_hbm.at[pl.ds(gbase, E_OUT)], idx_v.at[pl.ds(0, E_OUT)])

    def _gather(e, c):
        outbuf_v[pl.ds(e * L, L)] = plsc.load_gather(
            hist_v, [idx_v[pl.ds(e * L, L)]])
        return c
    lax.fori_loop(0, E_OUT // L, _gather, 0)
    pltpu.sync_copy(outbuf_v, out_hbm.at[pl.ds(gbase, E_OUT)])


@jax.jit
def _degree_gather(pos, tails):
    mesh = plsc.VectorSubcoreMesh(core_axis_name="c", subcore_axis_name="s")
    return pl.kernel(
        _sc_body,
        mesh=mesh,
        compiler_params=pltpu.CompilerParams(needs_layout_passes=False),
        out_type=jax.ShapeDtypeStruct((N_EDGES,), jnp.float32),
        scratch_types=[
            pltpu.VMEM((NPAD,), jnp.float32),
            pltpu.VMEM((E_HIST,), jnp.int32),
            pltpu.VMEM((NS, SLICE), jnp.float32),
            pltpu.VMEM((SLICE,), jnp.float32),
            pltpu.VMEM((E_OUT,), jnp.float32),
            pltpu.VMEM_SHARED((NS, NPAD), jnp.float32),
            pltpu.VMEM_SHARED((NPAD,), jnp.float32),
        ],
    )(pos, tails)


def kernel(z, edge_index, pos_edge_index):
    del z
    pos = pos_edge_index[1].astype(jnp.int32)
    tails = edge_index[1].astype(jnp.int32)
    return _degree_gather(pos, tails)


# speedup vs baseline: 51.4701x; 1.1600x over previous
import jax
import jax.numpy as jnp
from jax import lax
from jax.experimental import pallas as pl
from jax.experimental.pallas import tpu as pltpu
from jax.experimental.pallas import tpu_sc as plsc

N_NODES = 10000
N_EDGES = 320000
L = 16
NC = 2
NS = 16
NPAD = 10240
SLICE = NPAD // NS
E_HIST = N_EDGES // NS
E_OUT = N_EDGES // (NC * NS)


def _sc_body(pos_hbm, tail_hbm, out_hbm, hist_v, idx_v, staged_v, red_v,
             outbuf_v, rows_sh, deg_sh):
    cid = lax.axis_index("c")
    sid = lax.axis_index("s")
    wid = sid * NC + cid

    ones = jnp.ones((L,), jnp.float32)
    zeros = jnp.zeros((L,), jnp.float32)

    def _zero(i, c):
        hist_v[pl.ds(i * L, L)] = zeros
        return c
    lax.fori_loop(0, NPAD // L, _zero, 0)

    pltpu.sync_copy(pos_hbm.at[pl.ds(N_EDGES + sid * E_HIST, E_HIST)], idx_v)

    def _hist(e, c):
        plsc.addupdate_scatter(hist_v, [idx_v[pl.ds(e * L, L)]], ones)
        return c
    lax.fori_loop(0, E_HIST // L, _hist, 0)

    pltpu.sync_copy(hist_v, rows_sh.at[sid])
    plsc.subcore_barrier()

    col = sid * SLICE
    for r in range(NS):
        pltpu.sync_copy(rows_sh.at[r, pl.ds(col, SLICE)], staged_v.at[r])

    def _reduce(j, c):
        acc = staged_v[0, pl.ds(j * L, L)]
        for r in range(1, NS):
            acc = acc + staged_v[r, pl.ds(j * L, L)]
        red_v[pl.ds(j * L, L)] = acc
        return c
    lax.fori_loop(0, SLICE // L, _reduce, 0)
    pltpu.sync_copy(red_v, deg_sh.at[pl.ds(col, SLICE)])
    plsc.subcore_barrier()

    pltpu.sync_copy(deg_sh, hist_v)
    gbase = wid * E_OUT
    pltpu.sync_copy(tail_hbm.at[pl.ds(N_EDGES + gbase, E_OUT)],
                    idx_v.at[pl.ds(0, E_OUT)])

    def _gather(e, c):
        outbuf_v[pl.ds(e * L, L)] = plsc.load_gather(
            hist_v, [idx_v[pl.ds(e * L, L)]])
        return c
    lax.fori_loop(0, E_OUT // L, _gather, 0)
    pltpu.sync_copy(outbuf_v, out_hbm.at[pl.ds(gbase, E_OUT)])


@jax.jit
def _degree_gather(pos, tails):
    mesh = plsc.VectorSubcoreMesh(core_axis_name="c", subcore_axis_name="s")
    return pl.kernel(
        _sc_body,
        mesh=mesh,
        compiler_params=pltpu.CompilerParams(needs_layout_passes=False),
        out_type=jax.ShapeDtypeStruct((N_EDGES,), jnp.float32),
        scratch_types=[
            pltpu.VMEM((NPAD,), jnp.float32),
            pltpu.VMEM((E_HIST,), jnp.int32),
            pltpu.VMEM((NS, SLICE), jnp.float32),
            pltpu.VMEM((SLICE,), jnp.float32),
            pltpu.VMEM((E_OUT,), jnp.float32),
            pltpu.VMEM_SHARED((NS, NPAD), jnp.float32),
            pltpu.VMEM_SHARED((NPAD,), jnp.float32),
        ],
    )(pos, tails)


def kernel(z, edge_index, pos_edge_index):
    del z
    return _degree_gather(pos_edge_index.astype(jnp.int32).reshape(-1),
                          edge_index.astype(jnp.int32).reshape(-1))


# speedup vs baseline: 72.3445x; 1.4056x over previous
import jax
import jax.numpy as jnp
from jax import lax
from jax.experimental import pallas as pl
from jax.experimental.pallas import tpu as pltpu
from jax.experimental.pallas import tpu_sc as plsc

N_NODES = 10000
N_EDGES = 320000
L = 16
NC = 2
NS = 16
NPAD = 10240
SLICE = NPAD // NS
E_HIST = N_EDGES // NS
E_OUT = N_EDGES // (NC * NS)


def _sc_body(pos_hbm, tail_hbm, out_hbm, hist_v, idx_v, tails_v, staged_v,
             red_v, outbuf_v, rows_sh, deg_sh, sem_tails, sem_stage):
    cid = lax.axis_index("c")
    sid = lax.axis_index("s")
    wid = sid * NC + cid
    gbase = wid * E_OUT

    ones = jnp.ones((L,), jnp.float32)
    zeros = jnp.zeros((L,), jnp.float32)

    tails_cp = pltpu.async_copy(
        tail_hbm.at[pl.ds(N_EDGES + gbase, E_OUT)], tails_v, sem_tails)

    @plsc.parallel_loop(0, NPAD // L, unroll=8)
    def _zero(i):
        hist_v[pl.ds(i * L, L)] = zeros

    pltpu.sync_copy(pos_hbm.at[pl.ds(N_EDGES + sid * E_HIST, E_HIST)], idx_v)

    @plsc.parallel_loop(0, E_HIST // L, unroll=8)
    def _hist(e):
        plsc.addupdate_scatter(hist_v, [idx_v[pl.ds(e * L, L)]], ones)

    pltpu.sync_copy(hist_v, rows_sh.at[sid])
    plsc.subcore_barrier()

    col = sid * SLICE
    stage_cps = [
        pltpu.async_copy(rows_sh.at[r, pl.ds(col, SLICE)], staged_v.at[r],
                         sem_stage)
        for r in range(NS)
    ]
    for cp in stage_cps:
        cp.wait()

    @plsc.parallel_loop(0, SLICE // L, unroll=2)
    def _reduce(j):
        acc = staged_v[0, pl.ds(j * L, L)]
        for r in range(1, NS):
            acc = acc + staged_v[r, pl.ds(j * L, L)]
        red_v[pl.ds(j * L, L)] = acc

    pltpu.sync_copy(red_v, deg_sh.at[pl.ds(col, SLICE)])
    plsc.subcore_barrier()

    pltpu.sync_copy(deg_sh, hist_v)
    tails_cp.wait()

    @plsc.parallel_loop(0, E_OUT // L, unroll=8)
    def _gather(e):
        outbuf_v[pl.ds(e * L, L)] = plsc.load_gather(
            hist_v, [tails_v[pl.ds(e * L, L)]])

    pltpu.sync_copy(outbuf_v, out_hbm.at[pl.ds(gbase, E_OUT)])


@jax.jit
def _degree_gather(pos, tails):
    mesh = plsc.VectorSubcoreMesh(core_axis_name="c", subcore_axis_name="s")
    return pl.kernel(
        _sc_body,
        mesh=mesh,
        compiler_params=pltpu.CompilerParams(needs_layout_passes=False),
        out_type=jax.ShapeDtypeStruct((N_EDGES,), jnp.float32),
        scratch_types=[
            pltpu.VMEM((NPAD,), jnp.float32),
            pltpu.VMEM((E_HIST,), jnp.int32),
            pltpu.VMEM((E_OUT,), jnp.int32),
            pltpu.VMEM((NS, SLICE), jnp.float32),
            pltpu.VMEM((SLICE,), jnp.float32),
            pltpu.VMEM((E_OUT,), jnp.float32),
            pltpu.VMEM_SHARED((NS, NPAD), jnp.float32),
            pltpu.VMEM_SHARED((NPAD,), jnp.float32),
            pltpu.SemaphoreType.DMA,
            pltpu.SemaphoreType.DMA,
        ],
    )(pos, tails)


def kernel(z, edge_index, pos_edge_index):
    del z
    return _degree_gather(pos_edge_index.astype(jnp.int32).reshape(-1),
                          edge_index.astype(jnp.int32).reshape(-1))


# speedup vs baseline: 83.1842x; 1.1498x over previous
import jax
import jax.numpy as jnp
from jax import lax
from jax.experimental import pallas as pl
from jax.experimental.pallas import tpu as pltpu
from jax.experimental.pallas import tpu_sc as plsc

N_NODES = 10000
N_EDGES = 320000
L = 16
NC = 2
NS = 16
NPAD = 10240
SLICE = NPAD // NS
E_HIST = N_EDGES // NS
E_OUT = N_EDGES // (NC * NS)
TILE1 = 128
E_HIST_BUF = 20096
E_OUT_BUF = 10112


def _sc_body(pos_hbm, tail_hbm, out_hbm, hist_v, idx_v, tails_v, staged_v,
             red_v, outbuf_v, rows_sh, deg_sh, sem_tails, sem_stage):
    cid = lax.axis_index("c")
    sid = lax.axis_index("s")
    wid = sid * NC + cid
    gbase = wid * E_OUT

    ones = jnp.ones((L,), jnp.float32)
    zeros = jnp.zeros((L,), jnp.float32)

    start_t = jnp.minimum((gbase // TILE1) * TILE1, N_EDGES - E_OUT_BUF)
    start_t = pl.multiple_of(start_t, TILE1)
    off_t = gbase - start_t
    tails_cp = pltpu.async_copy(
        tail_hbm.at[:, pl.ds(start_t, E_OUT_BUF)], tails_v, sem_tails)

    @plsc.parallel_loop(0, NPAD // L, unroll=8)
    def _zero(i):
        hist_v[pl.ds(i * L, L)] = zeros

    start_h = jnp.minimum((sid * E_HIST // TILE1) * TILE1, N_EDGES - E_HIST_BUF)
    start_h = pl.multiple_of(start_h, TILE1)
    off_h = sid * E_HIST - start_h
    pltpu.sync_copy(pos_hbm.at[:, pl.ds(start_h, E_HIST_BUF)], idx_v)

    @plsc.parallel_loop(0, E_HIST // L, unroll=8)
    def _hist(e):
        plsc.addupdate_scatter(hist_v, [idx_v[1, pl.ds(off_h + e * L, L)]],
                               ones)

    pltpu.sync_copy(hist_v, rows_sh.at[sid])
    plsc.subcore_barrier()

    col = sid * SLICE
    stage_cps = [
        pltpu.async_copy(rows_sh.at[r, pl.ds(col, SLICE)], staged_v.at[r],
                         sem_stage)
        for r in range(NS)
    ]
    for cp in stage_cps:
        cp.wait()

    @plsc.parallel_loop(0, SLICE // L, unroll=2)
    def _reduce(j):
        acc = staged_v[0, pl.ds(j * L, L)]
        for r in range(1, NS):
            acc = acc + staged_v[r, pl.ds(j * L, L)]
        red_v[pl.ds(j * L, L)] = acc

    pltpu.sync_copy(red_v, deg_sh.at[pl.ds(col, SLICE)])
    plsc.subcore_barrier()

    pltpu.sync_copy(deg_sh, hist_v)
    tails_cp.wait()

    @plsc.parallel_loop(0, E_OUT // L, unroll=8)
    def _gather(e):
        outbuf_v[pl.ds(e * L, L)] = plsc.load_gather(
            hist_v, [tails_v[1, pl.ds(off_t + e * L, L)]])

    pltpu.sync_copy(outbuf_v, out_hbm.at[pl.ds(gbase, E_OUT)])


@jax.jit
def _degree_gather(pos, tails):
    mesh = plsc.VectorSubcoreMesh(core_axis_name="c", subcore_axis_name="s")
    return pl.kernel(
        _sc_body,
        mesh=mesh,
        compiler_params=pltpu.CompilerParams(needs_layout_passes=False),
        out_type=jax.ShapeDtypeStruct((N_EDGES,), jnp.float32),
        scratch_types=[
            pltpu.VMEM((NPAD,), jnp.float32),
            pltpu.VMEM((2, E_HIST_BUF), jnp.int32),
            pltpu.VMEM((2, E_OUT_BUF), jnp.int32),
            pltpu.VMEM((NS, SLICE), jnp.float32),
            pltpu.VMEM((SLICE,), jnp.float32),
            pltpu.VMEM((E_OUT,), jnp.float32),
            pltpu.VMEM_SHARED((NS, NPAD), jnp.float32),
            pltpu.VMEM_SHARED((NPAD,), jnp.float32),
            pltpu.SemaphoreType.DMA,
            pltpu.SemaphoreType.DMA,
        ],
    )(pos, tails)


def kernel(z, edge_index, pos_edge_index):
    del z
    return _degree_gather(pos_edge_index.astype(jnp.int32),
                          edge_index.astype(jnp.int32))


# speedup vs baseline: 84.7038x; 1.0183x over previous
import jax
import jax.numpy as jnp
from jax import lax
from jax.experimental import pallas as pl
from jax.experimental.pallas import tpu as pltpu
from jax.experimental.pallas import tpu_sc as plsc

N_NODES = 10000
N_EDGES = 320000
L = 16
NC = 2
NS = 16
NPAD = 10240
SLICE = NPAD // NS
E_HIST = N_EDGES // NS
E_OUT = N_EDGES // (NC * NS)
TILE1 = 128
E_HIST_BUF = 20096
E_OUT_BUF = 10112


def _sc_body(pos_hbm, tail_hbm, out_hbm, hist_v, idx_v, tails_v, staged_v,
             red_v, outbuf_v, rows_sh, deg_sh, sem_tails, sem_pos, sem_out,
             sem_stage):
    cid = lax.axis_index("c")
    sid = lax.axis_index("s")
    wid = sid * NC + cid
    gbase = wid * E_OUT

    ones = jnp.ones((L,), jnp.float32)
    zeros = jnp.zeros((L,), jnp.float32)

    start_t = jnp.minimum((gbase // TILE1) * TILE1, N_EDGES - E_OUT_BUF)
    start_t = pl.multiple_of(start_t, TILE1)
    off_t = gbase - start_t
    tails_cp = pltpu.async_copy(
        tail_hbm.at[:, pl.ds(start_t, E_OUT_BUF)], tails_v, sem_tails)

    start_h = jnp.minimum((sid * E_HIST // TILE1) * TILE1, N_EDGES - E_HIST_BUF)
    start_h = pl.multiple_of(start_h, TILE1)
    off_h = sid * E_HIST - start_h
    HC0 = 10112
    pos_cp0 = pltpu.async_copy(
        pos_hbm.at[:, pl.ds(start_h, HC0)], idx_v.at[:, pl.ds(0, HC0)],
        sem_pos)
    pos_cp1 = pltpu.async_copy(
        pos_hbm.at[:, pl.ds(start_h + HC0, E_HIST_BUF - HC0)],
        idx_v.at[:, pl.ds(HC0, E_HIST_BUF - HC0)], sem_pos)

    @plsc.parallel_loop(0, NPAD // L, unroll=8)
    def _zero(i):
        hist_v[pl.ds(i * L, L)] = zeros

    HSEG = E_HIST // L // 2
    pos_cp0.wait()

    @plsc.parallel_loop(0, HSEG, unroll=5)
    def _hist0(e):
        plsc.addupdate_scatter(hist_v, [idx_v[1, pl.ds(off_h + e * L, L)]],
                               ones)
    pos_cp1.wait()

    @plsc.parallel_loop(0, HSEG, unroll=5)
    def _hist1(e):
        plsc.addupdate_scatter(
            hist_v, [idx_v[1, pl.ds(off_h + (HSEG + e) * L, L)]], ones)

    pltpu.sync_copy(hist_v, rows_sh.at[sid])
    plsc.subcore_barrier()

    col = sid * SLICE
    stage_cps = [
        pltpu.async_copy(rows_sh.at[r, pl.ds(col, SLICE)], staged_v.at[r],
                         sem_stage)
        for r in range(NS)
    ]
    for cp in stage_cps:
        cp.wait()

    @plsc.parallel_loop(0, SLICE // L, unroll=2)
    def _reduce(j):
        acc = staged_v[0, pl.ds(j * L, L)]
        for r in range(1, NS):
            acc = acc + staged_v[r, pl.ds(j * L, L)]
        red_v[pl.ds(j * L, L)] = acc

    pltpu.sync_copy(red_v, deg_sh.at[pl.ds(col, SLICE)])
    plsc.subcore_barrier()

    pltpu.sync_copy(deg_sh, hist_v)
    tails_cp.wait()
    GSEG0 = 320
    GSEG1 = 305

    @plsc.parallel_loop(0, GSEG0, unroll=8)
    def _gather0(e):
        outbuf_v[pl.ds(e * L, L)] = plsc.load_gather(
            hist_v, [tails_v[1, pl.ds(off_t + e * L, L)]])

    out_cp0 = pltpu.async_copy(
        outbuf_v.at[pl.ds(0, GSEG0 * L)],
        out_hbm.at[pl.ds(gbase, GSEG0 * L)], sem_out)

    @plsc.parallel_loop(0, GSEG1, unroll=5)
    def _gather1(e):
        outbuf_v[pl.ds((GSEG0 + e) * L, L)] = plsc.load_gather(
            hist_v, [tails_v[1, pl.ds(off_t + (GSEG0 + e) * L, L)]])

    out_cp1 = pltpu.async_copy(
        outbuf_v.at[pl.ds(GSEG0 * L, GSEG1 * L)],
        out_hbm.at[pl.ds(gbase + GSEG0 * L, GSEG1 * L)], sem_out)
    out_cp0.wait()
    out_cp1.wait()


@jax.jit
def _degree_gather(pos, tails):
    mesh = plsc.VectorSubcoreMesh(core_axis_name="c", subcore_axis_name="s")
    return pl.kernel(
        _sc_body,
        mesh=mesh,
        compiler_params=pltpu.CompilerParams(needs_layout_passes=False),
        out_type=jax.ShapeDtypeStruct((N_EDGES,), jnp.float32),
        scratch_types=[
            pltpu.VMEM((NPAD,), jnp.float32),
            pltpu.VMEM((2, E_HIST_BUF), jnp.int32),
            pltpu.VMEM((2, E_OUT_BUF), jnp.int32),
            pltpu.VMEM((NS, SLICE), jnp.float32),
            pltpu.VMEM((SLICE,), jnp.float32),
            pltpu.VMEM((E_OUT,), jnp.float32),
            pltpu.VMEM_SHARED((NS, NPAD), jnp.float32),
            pltpu.VMEM_SHARED((NPAD,), jnp.float32),
            pltpu.SemaphoreType.DMA,
            pltpu.SemaphoreType.DMA,
            pltpu.SemaphoreType.DMA,
            pltpu.SemaphoreType.DMA,
        ],
    )(pos, tails)


def kernel(z, edge_index, pos_edge_index):
    del z
    return _degree_gather(pos_edge_index.astype(jnp.int32),
                          edge_index.astype(jnp.int32))
